# Initial kernel scaffold; baseline (speedup 1.0000x reference)
import jax
import jax.numpy as jnp
from jax.experimental import pallas as pl


def kernel(x, R):
    raise NotImplementedError("write your pallas kernel here")


# speedup vs baseline: 3.6200x; 3.6200x over previous
import jax
import jax.numpy as jnp
from jax import lax
from jax.experimental import pallas as pl
from jax.experimental.pallas import tpu as pltpu

_N_ITER = 50
_D = 256
_HI = lax.Precision.HIGHEST


def _dotT(a, b):
    return lax.dot_general(a, b, (((0,), (0,)), ((), ())),
                           preferred_element_type=jnp.float32, precision=_HI)


def _dot(a, b):
    return lax.dot_general(a, b, (((1,), (0,)), ((), ())),
                           preferred_element_type=jnp.float32, precision=_HI)


def _stats_kernel(x_ref, gram_ref, csum_ref):
    j = pl.program_id(1)

    @pl.when(j == 0)
    def _():
        gram_ref[...] = jnp.zeros_like(gram_ref)
        csum_ref[...] = jnp.zeros_like(csum_ref)

    xb = x_ref[...]
    gram_ref[...] += _dotT(xb, xb)[None]
    s = jnp.sum(xb, axis=0, keepdims=True)
    csum_ref[...] += jnp.broadcast_to(s[None], csum_ref.shape)


def _eig_kernel(n_rows, gram_ref, csum_ref, rt_ref, w_ref, mean_ref, a_scr):
    n = jnp.float32(n_rows)
    gram = gram_ref[0] + gram_ref[1]
    mean = (csum_ref[0, 0:1] + csum_ref[1, 0:1]) / n
    mmT = _dotT(mean, mean)
    a_scr[...] = gram / n - mmT
    w_ref[...] = jnp.zeros_like(w_ref)
    mean_ref[...] = jnp.broadcast_to(mean, mean_ref.shape)

    def outer_body(i, _):
        a = a_scr[...]
        v0 = rt_ref[pl.ds(i, 1), :]

        def power(_, v):
            v = _dot(v, a)
            return v * lax.rsqrt(jnp.sum(v * v, keepdims=True))

        v = lax.fori_loop(0, _N_ITER, power, v0)
        w = _dot(v, a)
        l = jnp.sqrt(jnp.sum(w * w, keepdims=True))
        vvT = _dotT(v, v)
        a_scr[...] = a - l * vvT
        w_ref[...] += vvT * lax.rsqrt(l)
        return 0

    lax.fori_loop(0, _D, outer_body, 0)


def _apply_kernel(x_ref, w_ref, mean_ref, o_ref):
    xc = x_ref[...] - mean_ref[0:1]
    o_ref[...] = _dot(xc, w_ref[...])


def kernel(x, R):
    n, d = x.shape
    assert d == _D
    block_m = 4096
    nb = n // (2 * block_m)

    gram_p, csum_p = pl.pallas_call(
        _stats_kernel,
        grid=(2, nb),
        in_specs=[pl.BlockSpec((block_m, d), lambda i, j: (i * nb + j, 0))],
        out_specs=[
            pl.BlockSpec((1, d, d), lambda i, j: (i, 0, 0)),
            pl.BlockSpec((1, 8, d), lambda i, j: (i, 0, 0)),
        ],
        out_shape=[
            jax.ShapeDtypeStruct((2, d, d), jnp.float32),
            jax.ShapeDtypeStruct((2, 8, d), jnp.float32),
        ],
        compiler_params=pltpu.CompilerParams(
            dimension_semantics=("parallel", "arbitrary")),
    )(x)

    import functools
    w, mean8 = pl.pallas_call(
        functools.partial(_eig_kernel, n),
        in_specs=[
            pl.BlockSpec((2, d, d), lambda: (0, 0, 0)),
            pl.BlockSpec((2, 8, d), lambda: (0, 0, 0)),
            pl.BlockSpec((d, d), lambda: (0, 0)),
        ],
        out_specs=[
            pl.BlockSpec((d, d), lambda: (0, 0)),
            pl.BlockSpec((8, d), lambda: (0, 0)),
        ],
        out_shape=[
            jax.ShapeDtypeStruct((d, d), jnp.float32),
            jax.ShapeDtypeStruct((8, d), jnp.float32),
        ],
        scratch_shapes=[pltpu.VMEM((d, d), jnp.float32)],
    )(gram_p, csum_p, R.T)

    out = pl.pallas_call(
        _apply_kernel,
        grid=(2, nb),
        in_specs=[
            pl.BlockSpec((block_m, d), lambda i, j: (i * nb + j, 0)),
            pl.BlockSpec((d, d), lambda i, j: (0, 0)),
            pl.BlockSpec((8, d), lambda i, j: (0, 0)),
        ],
        out_specs=pl.BlockSpec((block_m, d), lambda i, j: (i * nb + j, 0)),
        out_shape=jax.ShapeDtypeStruct((n, d), jnp.float32),
        compiler_params=pltpu.CompilerParams(
            dimension_semantics=("parallel", "arbitrary")),
    )(x, w, mean8)
    return out


# speedup vs baseline: 7.1245x; 1.9681x over previous
import jax
import jax.numpy as jnp
from jax import lax
from jax.experimental import pallas as pl
from jax.experimental.pallas import tpu as pltpu

_N_ITER = 50
_D = 256
_HI = lax.Precision.HIGHEST


def _dotT(a, b):
    return lax.dot_general(a, b, (((0,), (0,)), ((), ())),
                           preferred_element_type=jnp.float32, precision=_HI)


def _dot(a, b):
    return lax.dot_general(a, b, (((1,), (0,)), ((), ())),
                           preferred_element_type=jnp.float32, precision=_HI)


def _stats_kernel(x_ref, gram_ref, csum_ref):
    j = pl.program_id(1)

    @pl.when(j == 0)
    def _():
        gram_ref[...] = jnp.zeros_like(gram_ref)
        csum_ref[...] = jnp.zeros_like(csum_ref)

    xb = x_ref[...]
    gram_ref[...] += _dotT(xb, xb)[None]
    s = jnp.sum(xb, axis=0, keepdims=True)
    csum_ref[...] += jnp.broadcast_to(s[None], csum_ref.shape)


def _eig_kernel(n_rows, gram_ref, csum_ref, rt_ref, w_ref, mean_ref, a_scr):
    n = jnp.float32(n_rows)
    gram = gram_ref[0] + gram_ref[1]
    mean = (csum_ref[0, 0:1] + csum_ref[1, 0:1]) / n
    mmT = _dotT(mean, mean)
    a_scr[...] = gram / n - mmT
    w_ref[...] = jnp.zeros_like(w_ref)
    mean_ref[...] = jnp.broadcast_to(mean, mean_ref.shape)

    def outer_body(i, _):
        a = a_scr[...]
        v = rt_ref[pl.ds(i, 1), :]

        b = _dot(a, a)
        for k in range(_N_ITER // 2):
            v = _dot(v, b)
            if k % 3 == 2:
                v = v * lax.rsqrt(jnp.sum(v * v, keepdims=True))
        v = v * lax.rsqrt(jnp.sum(v * v, keepdims=True))
        w = _dot(v, a)
        l = jnp.sqrt(jnp.sum(w * w, keepdims=True))
        vvT = _dotT(v, v)
        a_scr[...] = a - l * vvT
        w_ref[...] += vvT * lax.rsqrt(l)
        return 0

    lax.fori_loop(0, _D, outer_body, 0)


def _apply_kernel(x_ref, w_ref, mean_ref, o_ref):
    xc = x_ref[...] - mean_ref[0:1]
    o_ref[...] = _dot(xc, w_ref[...])


def kernel(x, R):
    n, d = x.shape
    assert d == _D
    block_m = 4096
    nb = n // (2 * block_m)

    gram_p, csum_p = pl.pallas_call(
        _stats_kernel,
        grid=(2, nb),
        in_specs=[pl.BlockSpec((block_m, d), lambda i, j: (i * nb + j, 0))],
        out_specs=[
            pl.BlockSpec((1, d, d), lambda i, j: (i, 0, 0)),
            pl.BlockSpec((1, 8, d), lambda i, j: (i, 0, 0)),
        ],
        out_shape=[
            jax.ShapeDtypeStruct((2, d, d), jnp.float32),
            jax.ShapeDtypeStruct((2, 8, d), jnp.float32),
        ],
        compiler_params=pltpu.CompilerParams(
            dimension_semantics=("parallel", "arbitrary")),
    )(x)

    import functools
    w, mean8 = pl.pallas_call(
        functools.partial(_eig_kernel, n),
        in_specs=[
            pl.BlockSpec((2, d, d), lambda: (0, 0, 0)),
            pl.BlockSpec((2, 8, d), lambda: (0, 0, 0)),
            pl.BlockSpec((d, d), lambda: (0, 0)),
        ],
        out_specs=[
            pl.BlockSpec((d, d), lambda: (0, 0)),
            pl.BlockSpec((8, d), lambda: (0, 0)),
        ],
        out_shape=[
            jax.ShapeDtypeStruct((d, d), jnp.float32),
            jax.ShapeDtypeStruct((8, d), jnp.float32),
        ],
        scratch_shapes=[pltpu.VMEM((d, d), jnp.float32)],
    )(gram_p, csum_p, R.T)

    out = pl.pallas_call(
        _apply_kernel,
        grid=(2, nb),
        in_specs=[
            pl.BlockSpec((block_m, d), lambda i, j: (i * nb + j, 0)),
            pl.BlockSpec((d, d), lambda i, j: (0, 0)),
            pl.BlockSpec((8, d), lambda i, j: (0, 0)),
        ],
        out_specs=pl.BlockSpec((block_m, d), lambda i, j: (i * nb + j, 0)),
        out_shape=jax.ShapeDtypeStruct((n, d), jnp.float32),
        compiler_params=pltpu.CompilerParams(
            dimension_semantics=("parallel", "arbitrary")),
    )(x, w, mean8)
    return out


# speedup vs baseline: 11.9094x; 1.6716x over previous
import jax
import jax.numpy as jnp
from jax import lax
from jax.experimental import pallas as pl
from jax.experimental.pallas import tpu as pltpu

_N_ITER = 50
_D = 256
_HI = lax.Precision.HIGHEST


def _dotT(a, b):
    return lax.dot_general(a, b, (((0,), (0,)), ((), ())),
                           preferred_element_type=jnp.float32, precision=_HI)


def _dot(a, b, precision=_HI):
    return lax.dot_general(a, b, (((1,), (0,)), ((), ())),
                           preferred_element_type=jnp.float32,
                           precision=precision)


def _stats_kernel(x_ref, gram_ref, csum_ref):
    j = pl.program_id(1)

    @pl.when(j == 0)
    def _():
        gram_ref[...] = jnp.zeros_like(gram_ref)
        csum_ref[...] = jnp.zeros_like(csum_ref)

    xb = x_ref[...]
    gram_ref[...] += _dotT(xb, xb)[None]
    s = jnp.sum(xb, axis=0, keepdims=True)
    csum_ref[...] += jnp.broadcast_to(s[None], csum_ref.shape)


def _eig_kernel(n_rows, gram_ref, csum_ref, rt_ref, w_ref, mean_ref, a_scr):
    n = jnp.float32(n_rows)
    gram = gram_ref[0] + gram_ref[1]
    mean = (csum_ref[0, 0:1] + csum_ref[1, 0:1]) / n
    mmT = _dotT(mean, mean)
    a_scr[...] = gram / n - mmT
    w_ref[...] = jnp.zeros_like(w_ref)
    mean_ref[...] = jnp.broadcast_to(mean, mean_ref.shape)

    def outer_body(i, _):
        a = a_scr[...]
        v = rt_ref[pl.ds(i, 1), :]

        b = _dot(a, a)
        for k in range(_N_ITER // 2):
            v = _dot(v, b, lax.Precision.DEFAULT)
            if k % 3 == 2:
                v = v * lax.rsqrt(jnp.sum(v * v, keepdims=True))
        v = v * lax.rsqrt(jnp.sum(v * v, keepdims=True))
        w = _dot(v, a)
        l = jnp.sqrt(jnp.sum(w * w, keepdims=True))
        vvT = _dotT(v, v)
        a_scr[...] = a - l * vvT
        w_ref[...] += vvT * lax.rsqrt(l)
        return 0

    lax.fori_loop(0, _D, outer_body, 0)


def _apply_kernel(x_ref, w_ref, mean_ref, o_ref):
    xc = x_ref[...] - mean_ref[0:1]
    o_ref[...] = _dot(xc, w_ref[...])


def kernel(x, R):
    n, d = x.shape
    assert d == _D
    block_m = 4096
    nb = n // (2 * block_m)

    gram_p, csum_p = pl.pallas_call(
        _stats_kernel,
        grid=(2, nb),
        in_specs=[pl.BlockSpec((block_m, d), lambda i, j: (i * nb + j, 0))],
        out_specs=[
            pl.BlockSpec((1, d, d), lambda i, j: (i, 0, 0)),
            pl.BlockSpec((1, 8, d), lambda i, j: (i, 0, 0)),
        ],
        out_shape=[
            jax.ShapeDtypeStruct((2, d, d), jnp.float32),
            jax.ShapeDtypeStruct((2, 8, d), jnp.float32),
        ],
        compiler_params=pltpu.CompilerParams(
            dimension_semantics=("parallel", "arbitrary")),
    )(x)

    import functools
    w, mean8 = pl.pallas_call(
        functools.partial(_eig_kernel, n),
        in_specs=[
            pl.BlockSpec((2, d, d), lambda: (0, 0, 0)),
            pl.BlockSpec((2, 8, d), lambda: (0, 0, 0)),
            pl.BlockSpec((d, d), lambda: (0, 0)),
        ],
        out_specs=[
            pl.BlockSpec((d, d), lambda: (0, 0)),
            pl.BlockSpec((8, d), lambda: (0, 0)),
        ],
        out_shape=[
            jax.ShapeDtypeStruct((d, d), jnp.float32),
            jax.ShapeDtypeStruct((8, d), jnp.float32),
        ],
        scratch_shapes=[pltpu.VMEM((d, d), jnp.float32)],
    )(gram_p, csum_p, R.T)

    out = pl.pallas_call(
        _apply_kernel,
        grid=(2, nb),
        in_specs=[
            pl.BlockSpec((block_m, d), lambda i, j: (i * nb + j, 0)),
            pl.BlockSpec((d, d), lambda i, j: (0, 0)),
            pl.BlockSpec((8, d), lambda i, j: (0, 0)),
        ],
        out_specs=pl.BlockSpec((block_m, d), lambda i, j: (i * nb + j, 0)),
        out_shape=jax.ShapeDtypeStruct((n, d), jnp.float32),
        compiler_params=pltpu.CompilerParams(
            dimension_semantics=("parallel", "arbitrary")),
    )(x, w, mean8)
    return out


# speedup vs baseline: 22.5129x; 1.8903x over previous
import jax
import jax.numpy as jnp
from jax import lax
from jax.experimental import pallas as pl
from jax.experimental.pallas import tpu as pltpu

_N_ITER = 50
_D = 256
_HI = lax.Precision.HIGHEST


def _dotT(a, b):
    return lax.dot_general(a, b, (((0,), (0,)), ((), ())),
                           preferred_element_type=jnp.float32, precision=_HI)


def _dot(a, b, precision=_HI):
    return lax.dot_general(a, b, (((1,), (0,)), ((), ())),
                           preferred_element_type=jnp.float32,
                           precision=precision)


def _stats_kernel(x_ref, gram_ref, csum_ref):
    j = pl.program_id(1)

    @pl.when(j == 0)
    def _():
        gram_ref[...] = jnp.zeros_like(gram_ref)
        csum_ref[...] = jnp.zeros_like(csum_ref)

    xb = x_ref[...]
    gram_ref[...] += _dotT(xb, xb)[None]
    s = jnp.sum(xb, axis=0, keepdims=True)
    csum_ref[...] += jnp.broadcast_to(s[None], csum_ref.shape)


def _eig_kernel(n_rows, gram_ref, csum_ref, rt_ref, w_ref, mean_ref, a_scr):
    n = jnp.float32(n_rows)
    gram = gram_ref[0] + gram_ref[1]
    mean = (csum_ref[0, 0:1] + csum_ref[1, 0:1]) / n
    mmT = _dotT(mean, mean)
    a_scr[...] = gram / n - mmT
    w_ref[...] = jnp.zeros_like(w_ref)
    mean_ref[...] = jnp.broadcast_to(mean, mean_ref.shape)

    def _nrm(v):
        return v * lax.rsqrt(jnp.sum(v * v, keepdims=True))

    def outer_body(i, _):
        a = a_scr[...]
        v = rt_ref[pl.ds(i, 1), :]

        df = lax.Precision.DEFAULT
        a2 = _dot(a, a, df)
        a4 = _dot(a2, a2, df)
        a8 = _dot(a4, a4, df)
        a16 = _dot(a8, a8, df)
        v = _nrm(_dot(v, a2, df))
        v = _nrm(_dot(v, a16, df))
        v = _nrm(_dot(v, a16, df))
        v = _nrm(_dot(v, a16, df))
        w = _dot(v, a)
        l = jnp.sqrt(jnp.sum(w * w, keepdims=True))
        vvT = _dotT(v, v)
        a_scr[...] = a - l * vvT
        w_ref[...] += vvT * lax.rsqrt(l)
        return 0

    lax.fori_loop(0, _D, outer_body, 0)


def _apply_kernel(x_ref, w_ref, mean_ref, o_ref):
    xc = x_ref[...] - mean_ref[0:1]
    o_ref[...] = _dot(xc, w_ref[...])


def kernel(x, R):
    n, d = x.shape
    assert d == _D
    block_m = 4096
    nb = n // (2 * block_m)

    gram_p, csum_p = pl.pallas_call(
        _stats_kernel,
        grid=(2, nb),
        in_specs=[pl.BlockSpec((block_m, d), lambda i, j: (i * nb + j, 0))],
        out_specs=[
            pl.BlockSpec((1, d, d), lambda i, j: (i, 0, 0)),
            pl.BlockSpec((1, 8, d), lambda i, j: (i, 0, 0)),
        ],
        out_shape=[
            jax.ShapeDtypeStruct((2, d, d), jnp.float32),
            jax.ShapeDtypeStruct((2, 8, d), jnp.float32),
        ],
        compiler_params=pltpu.CompilerParams(
            dimension_semantics=("parallel", "arbitrary")),
    )(x)

    import functools
    w, mean8 = pl.pallas_call(
        functools.partial(_eig_kernel, n),
        in_specs=[
            pl.BlockSpec((2, d, d), lambda: (0, 0, 0)),
            pl.BlockSpec((2, 8, d), lambda: (0, 0, 0)),
            pl.BlockSpec((d, d), lambda: (0, 0)),
        ],
        out_specs=[
            pl.BlockSpec((d, d), lambda: (0, 0)),
            pl.BlockSpec((8, d), lambda: (0, 0)),
        ],
        out_shape=[
            jax.ShapeDtypeStruct((d, d), jnp.float32),
            jax.ShapeDtypeStruct((8, d), jnp.float32),
        ],
        scratch_shapes=[pltpu.VMEM((d, d), jnp.float32)],
    )(gram_p, csum_p, R.T)

    out = pl.pallas_call(
        _apply_kernel,
        grid=(2, nb),
        in_specs=[
            pl.BlockSpec((block_m, d), lambda i, j: (i * nb + j, 0)),
            pl.BlockSpec((d, d), lambda i, j: (0, 0)),
            pl.BlockSpec((8, d), lambda i, j: (0, 0)),
        ],
        out_specs=pl.BlockSpec((block_m, d), lambda i, j: (i * nb + j, 0)),
        out_shape=jax.ShapeDtypeStruct((n, d), jnp.float32),
        compiler_params=pltpu.CompilerParams(
            dimension_semantics=("parallel", "arbitrary")),
    )(x, w, mean8)
    return out


# speedup vs baseline: 24.2824x; 1.0786x over previous
import jax
import jax.numpy as jnp
from jax import lax
from jax.experimental import pallas as pl
from jax.experimental.pallas import tpu as pltpu

_N_ITER = 50
_D = 256
_HI = lax.Precision.HIGHEST


def _dotT(a, b):
    return lax.dot_general(a, b, (((0,), (0,)), ((), ())),
                           preferred_element_type=jnp.float32, precision=_HI)


def _dot(a, b, precision=_HI):
    return lax.dot_general(a, b, (((1,), (0,)), ((), ())),
                           preferred_element_type=jnp.float32,
                           precision=precision)


def _stats_kernel(x_ref, gram_ref, csum_ref):
    j = pl.program_id(1)

    @pl.when(j == 0)
    def _():
        gram_ref[...] = jnp.zeros_like(gram_ref)
        csum_ref[...] = jnp.zeros_like(csum_ref)

    xb = x_ref[...]
    gram_ref[...] += _dotT(xb, xb)[None]
    s = jnp.sum(xb, axis=0, keepdims=True)
    csum_ref[...] += jnp.broadcast_to(s[None], csum_ref.shape)


def _eig_kernel(n_rows, gram_ref, csum_ref, rt_ref, w_ref, mean_ref, a_scr):
    n = jnp.float32(n_rows)
    gram = gram_ref[0] + gram_ref[1]
    mean = (csum_ref[0, 0:1] + csum_ref[1, 0:1]) / n
    mmT = _dotT(mean, mean)
    a_scr[...] = gram / n - mmT
    w_ref[...] = jnp.zeros_like(w_ref)
    mean_ref[...] = jnp.broadcast_to(mean, mean_ref.shape)

    def _nrm(v):
        return v * lax.rsqrt(jnp.sum(v * v, keepdims=True))

    def outer_body(i, _):
        a = a_scr[...]
        v = rt_ref[pl.ds(i, 1), :]

        df = lax.Precision.DEFAULT
        a2 = _dot(a, a, df)
        a4 = _dot(a2, a2, df)
        a8 = _dot(a4, a4, df)
        a16 = _dot(a8, a8, df)
        v = _dot(v, a2, df)
        v = _nrm(_dot(v, a16, df))
        v = _dot(v, a16, df)
        v = _nrm(_dot(v, a16, df))
        w = _dot(v, a)
        l = jnp.sqrt(jnp.sum(w * w, keepdims=True))
        vvT = _dotT(v, v)
        a_scr[...] = a - l * vvT
        w_ref[...] += vvT * lax.rsqrt(l)
        return 0

    lax.fori_loop(0, _D, outer_body, 0)


def _apply_kernel(x_ref, w_ref, mean_ref, o_ref):
    xc = x_ref[...] - mean_ref[0:1]
    o_ref[...] = _dot(xc, w_ref[...])


def kernel(x, R):
    n, d = x.shape
    assert d == _D
    block_m = 4096
    nb = n // (2 * block_m)

    gram_p, csum_p = pl.pallas_call(
        _stats_kernel,
        grid=(2, nb),
        in_specs=[pl.BlockSpec((block_m, d), lambda i, j: (i * nb + j, 0))],
        out_specs=[
            pl.BlockSpec((1, d, d), lambda i, j: (i, 0, 0)),
            pl.BlockSpec((1, 8, d), lambda i, j: (i, 0, 0)),
        ],
        out_shape=[
            jax.ShapeDtypeStruct((2, d, d), jnp.float32),
            jax.ShapeDtypeStruct((2, 8, d), jnp.float32),
        ],
        compiler_params=pltpu.CompilerParams(
            dimension_semantics=("parallel", "arbitrary")),
    )(x)

    import functools
    w, mean8 = pl.pallas_call(
        functools.partial(_eig_kernel, n),
        in_specs=[
            pl.BlockSpec((2, d, d), lambda: (0, 0, 0)),
            pl.BlockSpec((2, 8, d), lambda: (0, 0, 0)),
            pl.BlockSpec((d, d), lambda: (0, 0)),
        ],
        out_specs=[
            pl.BlockSpec((d, d), lambda: (0, 0)),
            pl.BlockSpec((8, d), lambda: (0, 0)),
        ],
        out_shape=[
            jax.ShapeDtypeStruct((d, d), jnp.float32),
            jax.ShapeDtypeStruct((8, d), jnp.float32),
        ],
        scratch_shapes=[pltpu.VMEM((d, d), jnp.float32)],
    )(gram_p, csum_p, R.T)

    out = pl.pallas_call(
        _apply_kernel,
        grid=(2, nb),
        in_specs=[
            pl.BlockSpec((block_m, d), lambda i, j: (i * nb + j, 0)),
            pl.BlockSpec((d, d), lambda i, j: (0, 0)),
            pl.BlockSpec((8, d), lambda i, j: (0, 0)),
        ],
        out_specs=pl.BlockSpec((block_m, d), lambda i, j: (i * nb + j, 0)),
        out_shape=jax.ShapeDtypeStruct((n, d), jnp.float32),
        compiler_params=pltpu.CompilerParams(
            dimension_semantics=("parallel", "arbitrary")),
    )(x, w, mean8)
    return out
